# Initial kernel scaffold; baseline (speedup 1.0000x reference)
import jax
import jax.numpy as jnp
from jax.experimental import pallas as pl


def kernel(x, pos, edge_index, edge_attr, params):
    raise NotImplementedError("write your pallas kernel here")


# speedup vs baseline: 1.4676x; 1.4676x over previous
import jax
import jax.numpy as jnp
from jax import lax
from jax.experimental import pallas as pl
from jax.experimental.pallas import tpu as pltpu
from jax.experimental.pallas import tpu_sc as plsc

N = 2000
E = 50000
NLAYERS = 6
H = 128
ND = 64
ED = 64
ALPHA = 0.1
TEMP = 10.0

NC, NS, LANES = 2, 16, 16
NW = NC * NS
BURST = 128
NB = 13
EPT = NB * BURST
E_PAD = NW * EPT
EBLK = 512
EGRID = E_PAD // EBLK
TW = 128
AW = 128
HLANE = 128
NACC = 2048
NPT = NACC // NS
NBLK = 400
NGRID = N // NBLK
HHALF = (N * N // HLANE) // NC
HPR = 8192
HPASSES = 2
HOUT = HPASSES * HPR
HACC = HPR + BURST
HZPT = HACC // NS
HCPT = HPR // NS
EPT_H = E_PAD // NS
NBH = EPT_H // BURST

f32 = jnp.float32


def _sc_gather_body(table, gsrc, gdst, gs_out, gd_out, idx_v, rows_v, sem):
    c = lax.axis_index("c")
    s = lax.axis_index("s")
    wid = s * NC + c
    base = wid * EPT

    def run(idx_hbm, out_hbm):
        pltpu.sync_copy(idx_hbm.at[wid], idx_v)

        def step(j, carry):
            pltpu.async_copy(table.at[idx_v.at[j]], rows_v, sem).wait()
            pltpu.sync_copy(rows_v, out_hbm.at[pl.ds(base + j * BURST, BURST)])
            return carry

        lax.fori_loop(0, NB, step, 0)

    run(gsrc, gs_out)
    run(gdst, gd_out)


def _make_seg_scatter(mesh, widths, full=False):
    nw_ = len(widths)
    nbursts = NBH if full else NB

    def body(*refs):
        rows = refs[:nw_]
        sidx = refs[nw_]
        zrows = refs[nw_ + 1:2 * nw_ + 1]
        outs = refs[2 * nw_ + 1:3 * nw_ + 1]
        idx_v = refs[3 * nw_ + 1]
        bufs = refs[3 * nw_ + 2:4 * nw_ + 2]
        accs = refs[4 * nw_ + 2:5 * nw_ + 2]
        c = lax.axis_index("c")
        s = lax.axis_index("s")
        wid = s if full else s * NC + c
        ebase = wid * (EPT_H if full else EPT)

        def work():
            for z, acc in zip(zrows, accs):
                pltpu.sync_copy(z.at[pl.ds(s * NPT, NPT)],
                                acc.at[pl.ds(s * NPT, NPT)])
            plsc.subcore_barrier()
            pltpu.sync_copy(sidx.at[wid], idx_v)

            def step(j, carry):
                for r, buf, acc in zip(rows, bufs, accs):
                    pltpu.sync_copy(r.at[pl.ds(ebase + j * BURST, BURST)], buf)
                    pltpu.sync_copy(buf, acc.at[idx_v.at[j]], add=True)
                return carry

            lax.fori_loop(0, nbursts, step, 0)
            plsc.subcore_barrier()
            for acc, out in zip(accs, outs):
                dst = out.at[pl.ds(s * NPT, NPT)] if full \
                    else out.at[c, pl.ds(s * NPT, NPT)]
                pltpu.sync_copy(acc.at[pl.ds(s * NPT, NPT)], dst)

        if full:
            @pl.when(c == 0)
            def _():
                work()
        else:
            work()

    out_shape = ((NACC,) if full else (NC, NACC))
    return pl.kernel(
        body,
        out_type=tuple(jax.ShapeDtypeStruct(out_shape + (w,), f32)
                       for w in widths),
        mesh=mesh,
        scratch_types=[pltpu.VMEM((nbursts, BURST), jnp.int32)]
                      + [pltpu.VMEM((BURST, w), f32) for w in widths]
                      + [pltpu.VMEM_SHARED((NACC, w), f32) for w in widths],
    )


def _sc_heat_body(rows_hbm, hidx_hbm, zheat, out, idx_v, buf_v, acc):
    c = lax.axis_index("c")
    s = lax.axis_index("s")

    for ph in range(HPASSES):
        pltpu.sync_copy(zheat.at[pl.ds(s * HZPT, HZPT)],
                        acc.at[pl.ds(s * HZPT, HZPT)])
        plsc.subcore_barrier()
        pltpu.sync_copy(hidx_hbm.at[c, ph, s], idx_v)

        def step(j, carry):
            pltpu.sync_copy(rows_hbm.at[pl.ds(s * EPT_H + j * BURST, BURST)],
                            buf_v)
            pltpu.sync_copy(buf_v, acc.at[idx_v.at[j]], add=True)
            return carry

        lax.fori_loop(0, NBH, step, 0)
        plsc.subcore_barrier()
        pltpu.sync_copy(acc.at[pl.ds(s * HCPT, HCPT)],
                        out.at[c, pl.ds(ph * HPR + s * HCPT, HCPT)])
        plsc.subcore_barrier()


class _SCCalls:
    _cache = None

    @classmethod
    def get(cls):
        if cls._cache is None:
            mesh = plsc.VectorSubcoreMesh(
                core_axis_name="c", subcore_axis_name="s",
                num_cores=NC, num_subcores=NS)
            gather = pl.kernel(
                _sc_gather_body,
                out_type=(jax.ShapeDtypeStruct((E_PAD, TW), f32),
                          jax.ShapeDtypeStruct((E_PAD, TW), f32)),
                mesh=mesh,
                scratch_types=[pltpu.VMEM((NB, BURST), jnp.int32),
                               pltpu.VMEM((BURST, TW), f32),
                               pltpu.SemaphoreType.DMA])
            heat = pl.kernel(
                _sc_heat_body,
                out_type=jax.ShapeDtypeStruct((NC, HOUT, HLANE), f32),
                mesh=mesh,
                scratch_types=[pltpu.VMEM((NBH, BURST), jnp.int32),
                               pltpu.VMEM((BURST, HLANE), f32),
                               pltpu.VMEM_SHARED((HACC, HLANE), f32)])
            cls._cache = (gather, _make_seg_scatter(mesh, (H, AW)),
                          _make_seg_scatter(mesh, (HLANE,), full=True), heat)
        return cls._cache


def _silu(v):
    return v * (1.0 / (1.0 + jnp.exp(-v)))


def _node_embed_body(x, pos, win, bin_, tab_o):
    h0 = jnp.dot(x[:, :], win[:, :]) + bin_[:, :]
    li = lax.broadcasted_iota(jnp.int32, (NBLK, TW - ND), 1)
    p = pos[:, :]
    pack = jnp.where(li == 0, p[:, 0:1], jnp.where(li == 1, p[:, 1:2], 0.0))
    tab_o[:, :ND] = h0
    tab_o[:, ND:TW] = pack


def _edge_embed_body(ea, wein, bein, e_o):
    e_o[:, :] = jnp.dot(ea[:, :], wein[:, :]) + bein[:, :]


def _edge_body(gs, gd, e, w1hs, w1hd, w1e, w1d, b1, w2, b2,
               wx1, bx1, wx2, bx2, weue, weum, beu, m_o, aux_o, e_o):
    hs = gs[:, :ND]
    hd = gd[:, :ND]
    ev = e[:, :]
    rel = gs[:, ND:ND + 2] - gd[:, ND:ND + 2]
    dist2 = jnp.sum(rel * rel, axis=1, keepdims=True)
    t = (jnp.dot(hs, w1hs[:, :]) + jnp.dot(hd, w1hd[:, :]) +
         jnp.dot(ev, w1e[:, :]) + dist2 * w1d[:, :] + b1[:, :])
    m = _silu(jnp.dot(_silu(t), w2[:, :]) + b2[:, :])
    u = _silu(jnp.dot(m, wx1[:, :]) + bx1[:, :])
    wf = jnp.dot(u, wx2[:, :]) + bx2[:, :]
    w = jnp.tanh(wf[:, 0:1])
    relw = rel * w
    li = lax.broadcasted_iota(jnp.int32, (EBLK, AW), 1)
    pack = jnp.where(li == 0, relw[:, 0:1],
                     jnp.where(li == 1, relw[:, 1:2],
                               jnp.where(li == 2, jnp.float32(1.0),
                                         jnp.float32(0.0))))
    m_o[:, :] = m
    aux_o[:, :] = pack
    e_o[:, :] = ev + jnp.dot(ev, weue[:, :]) + jnp.dot(m, weum[:, :]) + beu[:, :]


def _node_body(tab, p0m, p1m, p0a, p1a, wh1h, wh1m, bh1, wh2, bh2, tab_o):
    hv = tab[:, :ND]
    coords = tab[:, ND:ND + 2]
    magg = p0m[:, :] + p1m[:, :]
    aux = p0a[:, :] + p1a[:, :]
    relsum = aux[:, 0:2]
    cnt = jnp.maximum(aux[:, 2:3], 1.0)
    cnew = coords + ALPHA * relsum / cnt
    g = _silu(jnp.dot(hv, wh1h[:, :]) + jnp.dot(magg, wh1m[:, :]) + bh1[:, :])
    hnew = hv + jnp.dot(g, wh2[:, :]) + bh2[:, :]
    li = lax.broadcasted_iota(jnp.int32, (NBLK, TW - ND), 1)
    pack = jnp.where(li == 0, cnew[:, 0:1], jnp.where(li == 1, cnew[:, 1:2], 0.0))
    tab_o[:, :ND] = hnew
    tab_o[:, ND:TW] = pack


def _final_body(gs, gd, e, src, dst, o1s, o1d, o1e, b1, o2, b2,
                pv_o, oh_o, hidx_o):
    i = pl.program_id(0)
    hs = gs[:, :ND]
    hd = gd[:, :ND]
    t = _silu(jnp.dot(hs, o1s[:, :]) + jnp.dot(hd, o1d[:, :]) +
              jnp.dot(e[:, :], o1e[:, :]) + b1[:, :])
    lg = jnp.dot(t, o2[:, :]) + b2[:, :]
    logit = TEMP * jnp.tanh(lg[:, 0:1])
    p = jnp.exp(logit)
    rows = i * EBLK + lax.broadcasted_iota(jnp.int32, (EBLK, 1), 0)
    p = jnp.where(rows < E, p, 0.0)
    li = lax.broadcasted_iota(jnp.int32, (EBLK, HLANE), 1)
    pv_o[:, :] = jnp.where(li == 0, p, 0.0)
    flat = src[:, :] * N + dst[:, :]
    hrow = lax.shift_right_logical(flat, 7)
    lane = lax.bitwise_and(flat, HLANE - 1)
    oh_o[:, :] = jnp.where(li == lane, p, 0.0)
    li8 = lax.broadcasted_iota(jnp.int32, (EBLK, 8), 1)
    acc_idx = jnp.zeros((EBLK, 8), jnp.int32)
    for k in range(NC * HPASSES):
        cc, ph = divmod(k, HPASSES)
        local = hrow - (cc * HHALF + ph * HPR)
        ok = (local >= 0) & (local < HPR)
        local = jnp.where(ok, local, HPR)
        acc_idx = jnp.where(li8 == k, local, acc_idx)
    hidx_o[:, :] = acc_idx


def _divide_body(hm, den, out):
    out[:, :] = hm[:, :] / jnp.maximum(den[:, :], 1e-30)


def _full(shape):
    return pl.BlockSpec(shape, lambda i: (0,) * len(shape))


def _eblk(width):
    return pl.BlockSpec((EBLK, width), lambda i: (i, 0))


def _nblk(width):
    return pl.BlockSpec((NBLK, width), lambda i: (i, 0))


_node_embed_call = pl.pallas_call(
    _node_embed_body,
    grid=(NGRID,),
    in_specs=[_nblk(2), _nblk(2), _full((2, ND)), _full((1, ND))],
    out_specs=_nblk(TW),
    out_shape=jax.ShapeDtypeStruct((N, TW), f32),
)

_edge_embed_call = pl.pallas_call(
    _edge_embed_body,
    grid=(EGRID,),
    in_specs=[_eblk(2), _full((2, ED)), _full((1, ED))],
    out_specs=_eblk(ED),
    out_shape=jax.ShapeDtypeStruct((E_PAD, ED), f32),
)

_edge_call = pl.pallas_call(
    _edge_body,
    grid=(EGRID,),
    in_specs=[_eblk(TW), _eblk(TW), _eblk(ED),
              _full((ND, H)), _full((ND, H)), _full((ED, H)), _full((1, H)),
              _full((1, H)), _full((H, H)), _full((1, H)),
              _full((H, H)), _full((1, H)), _full((H, 8)), _full((1, 8)),
              _full((ED, ED)), _full((H, ED)), _full((1, ED))],
    out_specs=[_eblk(H), _eblk(AW), _eblk(ED)],
    out_shape=[jax.ShapeDtypeStruct((E_PAD, H), f32),
               jax.ShapeDtypeStruct((E_PAD, AW), f32),
               jax.ShapeDtypeStruct((E_PAD, ED), f32)],
)

_node_call = pl.pallas_call(
    _node_body,
    grid=(NGRID,),
    in_specs=[_nblk(TW), _nblk(H), _nblk(H), _nblk(AW), _nblk(AW),
              _full((ND, H)), _full((H, H)), _full((1, H)),
              _full((H, ND)), _full((1, ND))],
    out_specs=_nblk(TW),
    out_shape=jax.ShapeDtypeStruct((N, TW), f32),
)

_final_call = pl.pallas_call(
    _final_body,
    grid=(EGRID,),
    in_specs=[_eblk(TW), _eblk(TW), _eblk(ED), _eblk(1), _eblk(1),
              _full((ND, H)), _full((ND, H)), _full((ED, H)), _full((1, H)),
              _full((H, 8)), _full((1, 8))],
    out_specs=[_eblk(HLANE), _eblk(HLANE), _eblk(8)],
    out_shape=[jax.ShapeDtypeStruct((E_PAD, HLANE), f32),
               jax.ShapeDtypeStruct((E_PAD, HLANE), f32),
               jax.ShapeDtypeStruct((E_PAD, 8), jnp.int32)],
)

_divide_call = pl.pallas_call(
    _divide_body,
    grid=(NGRID,),
    in_specs=[_nblk(N), _nblk(1)],
    out_specs=_nblk(N),
    out_shape=jax.ShapeDtypeStruct((N, N), f32),
)


def kernel(x, pos, edge_index, edge_attr, params):
    src = edge_index[0].astype(jnp.int32)
    dst = edge_index[1].astype(jnp.int32)
    pad = E_PAD - E
    src_g = jnp.concatenate([src, jnp.zeros((pad,), jnp.int32)])
    dst_g = jnp.concatenate([dst, jnp.zeros((pad,), jnp.int32)])
    src_s = jnp.concatenate([src, jnp.full((pad,), N, jnp.int32)])
    gsrc = src_g.reshape(NW, NB, BURST)
    gdst = dst_g.reshape(NW, NB, BURST)
    sidx = src_s.reshape(NW, NB, BURST)
    sidx_h = src_s.reshape(NS, NBH, BURST)
    ea = jnp.concatenate([edge_attr, jnp.zeros((pad, edge_attr.shape[1]), f32)])
    z_m = jnp.zeros((NACC, H), f32)
    z_aux = jnp.zeros((NACC, AW), f32)
    z_den = jnp.zeros((NACC, HLANE), f32)
    z_heat = jnp.zeros((HACC, HLANE), f32)

    _gather, _seg_scatter_m, _seg_scatter_p, _heat = _SCCalls.get()

    win, bin_ = params['node_in']
    wein, bein = params['edge_in']
    tab = _node_embed_call(x, pos, win, bin_.reshape(1, ND))
    e = _edge_embed_call(ea, wein, bein.reshape(1, ED))

    for i in range(NLAYERS):
        p = params[f'layer{i}']
        w1, b1 = p['We1']
        w2, b2 = p['We2']
        wx1, bx1 = p['Wx1']
        wx2, bx2 = p['Wx2']
        wh1, bh1 = p['Wh1']
        wh2, bh2 = p['Wh2']
        weu, beu = p['Weu']
        gs, gd = _gather(tab, gsrc, gdst)
        scat_m, scat_a, e = _edge_call(
            gs, gd, e,
            w1[:ND], w1[ND:2 * ND], w1[2 * ND:2 * ND + ED], w1[2 * ND + ED:],
            b1.reshape(1, H), w2, b2.reshape(1, H), wx1, bx1.reshape(1, H),
            jnp.pad(wx2, ((0, 0), (0, 7))), jnp.pad(bx2, (0, 7)).reshape(1, 8),
            weu[:ED], weu[ED:], beu.reshape(1, ED))
        pm, pa = _seg_scatter_m(scat_m, scat_a, sidx, z_m, z_aux)
        tab = _node_call(tab, pm[0, :N], pm[1, :N], pa[0, :N], pa[1, :N],
                         wh1[:ND], wh1[ND:], bh1.reshape(1, H),
                         wh2, bh2.reshape(1, ND))

    o1, ob1 = params['out1']
    o2, ob2 = params['out2']
    gs, gd = _gather(tab, gsrc, gdst)
    pvec, ohrows, hidx = _final_call(
        gs, gd, e, src_s.reshape(E_PAD, 1), dst_g.reshape(E_PAD, 1),
        o1[:ND], o1[ND:2 * ND], o1[2 * ND:], ob1.reshape(1, H),
        jnp.pad(o2, ((0, 0), (0, 7))),
        jnp.pad(ob2, (0, 7)).reshape(1, 8))
    den, = _seg_scatter_p(pvec, sidx_h, z_den)
    hidx_r = jnp.stack([hidx[:, k].reshape(NS, NBH, BURST)
                        for k in range(NC * HPASSES)])
    hidx_r = hidx_r.reshape(NC, HPASSES, NS, NBH, BURST)
    hm = _heat(ohrows, hidx_r, z_heat)
    hm2d = hm[:, :HHALF, :].reshape(N, N)
    return _divide_call(hm2d, den[:N, 0:1])


# speedup vs baseline: 1.5093x; 1.0284x over previous
import jax
import jax.numpy as jnp
from jax import lax
from jax.experimental import pallas as pl
from jax.experimental.pallas import tpu as pltpu
from jax.experimental.pallas import tpu_sc as plsc

N = 2000
E = 50000
NLAYERS = 6
H = 128
ND = 64
ED = 64
ALPHA = 0.1
TEMP = 10.0

NC, NS, LANES = 2, 16, 16
NW = NC * NS
BURST = 128
NB = 13
EPT = NB * BURST
E_PAD = NW * EPT
EBLK = 512
EGRID = E_PAD // EBLK
TW = 128
AW = 128
HLANE = 128
NACC = 2048
NPT = NACC // NS
NBLK = 400
NGRID = N // NBLK
HHALF = (N * N // HLANE) // NC
HPR = 8192
HPASSES = 2
HOUT = HPASSES * HPR
HACC = HPR + BURST
HZPT = HACC // NS
HCPT = HPR // NS
EPT_H = E_PAD // NS
NBH = EPT_H // BURST

f32 = jnp.float32


GCHUNK = 3


def _sc_gather_body(table, gsrc, gdst, gs_out, gd_out, idx_v, rows_v, sem):
    c = lax.axis_index("c")
    s = lax.axis_index("s")
    wid = s * NC + c
    base = wid * EPT

    def run(idx_hbm, out_hbm):
        pltpu.sync_copy(idx_hbm.at[wid], idx_v)

        def chunk(k, carry):
            j0 = k * GCHUNK
            descs = [
                pltpu.async_copy(table.at[idx_v.at[j0 + b]],
                                 rows_v.at[pl.ds(b * BURST, BURST)], sem)
                for b in range(GCHUNK)
            ]
            for d in descs:
                d.wait()
            pltpu.sync_copy(
                rows_v.at[pl.ds(0, GCHUNK * BURST)],
                out_hbm.at[pl.ds(base + j0 * BURST, GCHUNK * BURST)])
            return carry

        lax.fori_loop(0, NB // GCHUNK, chunk, 0)
        j_tail = (NB // GCHUNK) * GCHUNK
        pltpu.async_copy(table.at[idx_v.at[j_tail]],
                         rows_v.at[pl.ds(0, BURST)], sem).wait()
        pltpu.sync_copy(rows_v.at[pl.ds(0, BURST)],
                        out_hbm.at[pl.ds(base + j_tail * BURST, BURST)])

    run(gsrc, gs_out)
    run(gdst, gd_out)


def _make_seg_scatter(mesh, widths, full=False):
    nw_ = len(widths)
    nbursts = NBH if full else NB

    schunk = 2
    nchunks = nbursts // schunk
    tail = nbursts - nchunks * schunk

    def body(*refs):
        rows = refs[:nw_]
        sidx = refs[nw_]
        zrows = refs[nw_ + 1:2 * nw_ + 1]
        outs = refs[2 * nw_ + 1:3 * nw_ + 1]
        idx_v = refs[3 * nw_ + 1]
        sem = refs[3 * nw_ + 2]
        bufs = refs[3 * nw_ + 3:4 * nw_ + 3]
        accs = refs[4 * nw_ + 3:5 * nw_ + 3]
        c = lax.axis_index("c")
        s = lax.axis_index("s")
        wid = s if full else s * NC + c
        ebase = wid * (EPT_H if full else EPT)

        def work():
            for z, acc in zip(zrows, accs):
                pltpu.sync_copy(z.at[pl.ds(s * NPT, NPT)],
                                acc.at[pl.ds(s * NPT, NPT)])
            plsc.subcore_barrier()
            pltpu.sync_copy(sidx.at[wid], idx_v)

            def chunk(k, carry):
                j0 = k * schunk
                for r, buf, acc in zip(rows, bufs, accs):
                    pltpu.sync_copy(
                        r.at[pl.ds(ebase + j0 * BURST, schunk * BURST)], buf)
                    descs = [
                        pltpu.async_copy(buf.at[pl.ds(b * BURST, BURST)],
                                         acc.at[idx_v.at[j0 + b]], sem,
                                         add=True)
                        for b in range(schunk)
                    ]
                    for d in descs:
                        d.wait()
                return carry

            lax.fori_loop(0, nchunks, chunk, 0)
            for j in range(nchunks * schunk, nchunks * schunk + tail):
                for r, buf, acc in zip(rows, bufs, accs):
                    pltpu.sync_copy(r.at[pl.ds(ebase + j * BURST, BURST)],
                                    buf.at[pl.ds(0, BURST)])
                    pltpu.async_copy(buf.at[pl.ds(0, BURST)],
                                     acc.at[idx_v.at[j]], sem,
                                     add=True).wait()
            plsc.subcore_barrier()
            for acc, out in zip(accs, outs):
                dst = out.at[pl.ds(s * NPT, NPT)] if full \
                    else out.at[c, pl.ds(s * NPT, NPT)]
                pltpu.sync_copy(acc.at[pl.ds(s * NPT, NPT)], dst)

        if full:
            @pl.when(c == 0)
            def _():
                work()
        else:
            work()

    out_shape = ((NACC,) if full else (NC, NACC))
    return pl.kernel(
        body,
        out_type=tuple(jax.ShapeDtypeStruct(out_shape + (w,), f32)
                       for w in widths),
        mesh=mesh,
        scratch_types=[pltpu.VMEM((nbursts, BURST), jnp.int32),
                       pltpu.SemaphoreType.DMA]
                      + [pltpu.VMEM((schunk * BURST, w), f32) for w in widths]
                      + [pltpu.VMEM_SHARED((NACC, w), f32) for w in widths],
    )


HCHUNK = 2


def _sc_heat_body(rows_hbm, hidx_hbm, zheat, out, idx_v, buf_v, sem, acc):
    c = lax.axis_index("c")
    s = lax.axis_index("s")

    for ph in range(HPASSES):
        pltpu.sync_copy(zheat.at[pl.ds(s * HZPT, HZPT)],
                        acc.at[pl.ds(s * HZPT, HZPT)])
        plsc.subcore_barrier()
        pltpu.sync_copy(hidx_hbm.at[c, ph, s], idx_v)

        def chunk(k, carry):
            j0 = k * HCHUNK
            pltpu.sync_copy(
                rows_hbm.at[pl.ds(s * EPT_H + j0 * BURST, HCHUNK * BURST)],
                buf_v)
            descs = [
                pltpu.async_copy(buf_v.at[pl.ds(b * BURST, BURST)],
                                 acc.at[idx_v.at[j0 + b]], sem, add=True)
                for b in range(HCHUNK)
            ]
            for d in descs:
                d.wait()
            return carry

        lax.fori_loop(0, NBH // HCHUNK, chunk, 0)
        plsc.subcore_barrier()
        pltpu.sync_copy(acc.at[pl.ds(s * HCPT, HCPT)],
                        out.at[c, pl.ds(ph * HPR + s * HCPT, HCPT)])
        plsc.subcore_barrier()


class _SCCalls:
    _cache = None

    @classmethod
    def get(cls):
        if cls._cache is None:
            mesh = plsc.VectorSubcoreMesh(
                core_axis_name="c", subcore_axis_name="s",
                num_cores=NC, num_subcores=NS)
            gather = pl.kernel(
                _sc_gather_body,
                out_type=(jax.ShapeDtypeStruct((E_PAD, TW), f32),
                          jax.ShapeDtypeStruct((E_PAD, TW), f32)),
                mesh=mesh,
                scratch_types=[pltpu.VMEM((NB, BURST), jnp.int32),
                               pltpu.VMEM((GCHUNK * BURST, TW), f32),
                               pltpu.SemaphoreType.DMA])
            heat = pl.kernel(
                _sc_heat_body,
                out_type=jax.ShapeDtypeStruct((NC, HOUT, HLANE), f32),
                mesh=mesh,
                scratch_types=[pltpu.VMEM((NBH, BURST), jnp.int32),
                               pltpu.VMEM((HCHUNK * BURST, HLANE), f32),
                               pltpu.SemaphoreType.DMA,
                               pltpu.VMEM_SHARED((HACC, HLANE), f32)])
            cls._cache = (gather, _make_seg_scatter(mesh, (H, AW)),
                          _make_seg_scatter(mesh, (HLANE,), full=True), heat)
        return cls._cache


def _silu(v):
    return v * (1.0 / (1.0 + jnp.exp(-v)))


def _node_embed_body(x, pos, win, bin_, tab_o):
    h0 = jnp.dot(x[:, :], win[:, :]) + bin_[:, :]
    li = lax.broadcasted_iota(jnp.int32, (NBLK, TW - ND), 1)
    p = pos[:, :]
    pack = jnp.where(li == 0, p[:, 0:1], jnp.where(li == 1, p[:, 1:2], 0.0))
    tab_o[:, :ND] = h0
    tab_o[:, ND:TW] = pack


def _edge_embed_body(ea, wein, bein, e_o):
    e_o[:, :] = jnp.dot(ea[:, :], wein[:, :]) + bein[:, :]


def _edge_body(gs, gd, e, w1hs, w1hd, w1e, w1d, b1, w2, b2,
               wx1, bx1, wx2, bx2, weue, weum, beu, m_o, aux_o, e_o):
    hs = gs[:, :ND]
    hd = gd[:, :ND]
    ev = e[:, :]
    rel = gs[:, ND:ND + 2] - gd[:, ND:ND + 2]
    dist2 = jnp.sum(rel * rel, axis=1, keepdims=True)
    t = (jnp.dot(hs, w1hs[:, :]) + jnp.dot(hd, w1hd[:, :]) +
         jnp.dot(ev, w1e[:, :]) + dist2 * w1d[:, :] + b1[:, :])
    m = _silu(jnp.dot(_silu(t), w2[:, :]) + b2[:, :])
    u = _silu(jnp.dot(m, wx1[:, :]) + bx1[:, :])
    wf = jnp.dot(u, wx2[:, :]) + bx2[:, :]
    w = jnp.tanh(wf[:, 0:1])
    relw = rel * w
    li = lax.broadcasted_iota(jnp.int32, (EBLK, AW), 1)
    pack = jnp.where(li == 0, relw[:, 0:1],
                     jnp.where(li == 1, relw[:, 1:2],
                               jnp.where(li == 2, jnp.float32(1.0),
                                         jnp.float32(0.0))))
    m_o[:, :] = m
    aux_o[:, :] = pack
    e_o[:, :] = ev + jnp.dot(ev, weue[:, :]) + jnp.dot(m, weum[:, :]) + beu[:, :]


def _node_body(tab, p0m, p1m, p0a, p1a, wh1h, wh1m, bh1, wh2, bh2, tab_o):
    hv = tab[:, :ND]
    coords = tab[:, ND:ND + 2]
    magg = p0m[:, :] + p1m[:, :]
    aux = p0a[:, :] + p1a[:, :]
    relsum = aux[:, 0:2]
    cnt = jnp.maximum(aux[:, 2:3], 1.0)
    cnew = coords + ALPHA * relsum / cnt
    g = _silu(jnp.dot(hv, wh1h[:, :]) + jnp.dot(magg, wh1m[:, :]) + bh1[:, :])
    hnew = hv + jnp.dot(g, wh2[:, :]) + bh2[:, :]
    li = lax.broadcasted_iota(jnp.int32, (NBLK, TW - ND), 1)
    pack = jnp.where(li == 0, cnew[:, 0:1], jnp.where(li == 1, cnew[:, 1:2], 0.0))
    tab_o[:, :ND] = hnew
    tab_o[:, ND:TW] = pack


def _final_body(gs, gd, e, src, dst, o1s, o1d, o1e, b1, o2, b2,
                pv_o, oh_o, hidx_o):
    i = pl.program_id(0)
    hs = gs[:, :ND]
    hd = gd[:, :ND]
    t = _silu(jnp.dot(hs, o1s[:, :]) + jnp.dot(hd, o1d[:, :]) +
              jnp.dot(e[:, :], o1e[:, :]) + b1[:, :])
    lg = jnp.dot(t, o2[:, :]) + b2[:, :]
    logit = TEMP * jnp.tanh(lg[:, 0:1])
    p = jnp.exp(logit)
    rows = i * EBLK + lax.broadcasted_iota(jnp.int32, (EBLK, 1), 0)
    p = jnp.where(rows < E, p, 0.0)
    li = lax.broadcasted_iota(jnp.int32, (EBLK, HLANE), 1)
    pv_o[:, :] = jnp.where(li == 0, p, 0.0)
    flat = src[:, :] * N + dst[:, :]
    hrow = lax.shift_right_logical(flat, 7)
    lane = lax.bitwise_and(flat, HLANE - 1)
    oh_o[:, :] = jnp.where(li == lane, p, 0.0)
    li8 = lax.broadcasted_iota(jnp.int32, (EBLK, 8), 1)
    acc_idx = jnp.zeros((EBLK, 8), jnp.int32)
    for k in range(NC * HPASSES):
        cc, ph = divmod(k, HPASSES)
        local = hrow - (cc * HHALF + ph * HPR)
        ok = (local >= 0) & (local < HPR)
        local = jnp.where(ok, local, HPR)
        acc_idx = jnp.where(li8 == k, local, acc_idx)
    hidx_o[:, :] = acc_idx


def _divide_body(hm, den, out):
    out[:, :] = hm[:, :] / jnp.maximum(den[:, :], 1e-30)


def _full(shape):
    return pl.BlockSpec(shape, lambda i: (0,) * len(shape))


def _eblk(width):
    return pl.BlockSpec((EBLK, width), lambda i: (i, 0))


def _nblk(width):
    return pl.BlockSpec((NBLK, width), lambda i: (i, 0))


_node_embed_call = pl.pallas_call(
    _node_embed_body,
    grid=(NGRID,),
    in_specs=[_nblk(2), _nblk(2), _full((2, ND)), _full((1, ND))],
    out_specs=_nblk(TW),
    out_shape=jax.ShapeDtypeStruct((N, TW), f32),
)

_edge_embed_call = pl.pallas_call(
    _edge_embed_body,
    grid=(EGRID,),
    in_specs=[_eblk(2), _full((2, ED)), _full((1, ED))],
    out_specs=_eblk(ED),
    out_shape=jax.ShapeDtypeStruct((E_PAD, ED), f32),
)

_edge_call = pl.pallas_call(
    _edge_body,
    grid=(EGRID,),
    in_specs=[_eblk(TW), _eblk(TW), _eblk(ED),
              _full((ND, H)), _full((ND, H)), _full((ED, H)), _full((1, H)),
              _full((1, H)), _full((H, H)), _full((1, H)),
              _full((H, H)), _full((1, H)), _full((H, 8)), _full((1, 8)),
              _full((ED, ED)), _full((H, ED)), _full((1, ED))],
    out_specs=[_eblk(H), _eblk(AW), _eblk(ED)],
    out_shape=[jax.ShapeDtypeStruct((E_PAD, H), f32),
               jax.ShapeDtypeStruct((E_PAD, AW), f32),
               jax.ShapeDtypeStruct((E_PAD, ED), f32)],
)

_node_call = pl.pallas_call(
    _node_body,
    grid=(NGRID,),
    in_specs=[_nblk(TW), _nblk(H), _nblk(H), _nblk(AW), _nblk(AW),
              _full((ND, H)), _full((H, H)), _full((1, H)),
              _full((H, ND)), _full((1, ND))],
    out_specs=_nblk(TW),
    out_shape=jax.ShapeDtypeStruct((N, TW), f32),
)

_final_call = pl.pallas_call(
    _final_body,
    grid=(EGRID,),
    in_specs=[_eblk(TW), _eblk(TW), _eblk(ED), _eblk(1), _eblk(1),
              _full((ND, H)), _full((ND, H)), _full((ED, H)), _full((1, H)),
              _full((H, 8)), _full((1, 8))],
    out_specs=[_eblk(HLANE), _eblk(HLANE), _eblk(8)],
    out_shape=[jax.ShapeDtypeStruct((E_PAD, HLANE), f32),
               jax.ShapeDtypeStruct((E_PAD, HLANE), f32),
               jax.ShapeDtypeStruct((E_PAD, 8), jnp.int32)],
)

_divide_call = pl.pallas_call(
    _divide_body,
    grid=(NGRID,),
    in_specs=[_nblk(N), _nblk(1)],
    out_specs=_nblk(N),
    out_shape=jax.ShapeDtypeStruct((N, N), f32),
)


def kernel(x, pos, edge_index, edge_attr, params):
    src = edge_index[0].astype(jnp.int32)
    dst = edge_index[1].astype(jnp.int32)
    pad = E_PAD - E
    src_g = jnp.concatenate([src, jnp.zeros((pad,), jnp.int32)])
    dst_g = jnp.concatenate([dst, jnp.zeros((pad,), jnp.int32)])
    src_s = jnp.concatenate([src, jnp.full((pad,), N, jnp.int32)])
    gsrc = src_g.reshape(NW, NB, BURST)
    gdst = dst_g.reshape(NW, NB, BURST)
    sidx = src_s.reshape(NW, NB, BURST)
    sidx_h = src_s.reshape(NS, NBH, BURST)
    ea = jnp.concatenate([edge_attr, jnp.zeros((pad, edge_attr.shape[1]), f32)])
    z_m = jnp.zeros((NACC, H), f32)
    z_aux = jnp.zeros((NACC, AW), f32)
    z_den = jnp.zeros((NACC, HLANE), f32)
    z_heat = jnp.zeros((HACC, HLANE), f32)

    _gather, _seg_scatter_m, _seg_scatter_p, _heat = _SCCalls.get()

    win, bin_ = params['node_in']
    wein, bein = params['edge_in']
    tab = _node_embed_call(x, pos, win, bin_.reshape(1, ND))
    e = _edge_embed_call(ea, wein, bein.reshape(1, ED))

    for i in range(NLAYERS):
        p = params[f'layer{i}']
        w1, b1 = p['We1']
        w2, b2 = p['We2']
        wx1, bx1 = p['Wx1']
        wx2, bx2 = p['Wx2']
        wh1, bh1 = p['Wh1']
        wh2, bh2 = p['Wh2']
        weu, beu = p['Weu']
        gs, gd = _gather(tab, gsrc, gdst)
        scat_m, scat_a, e = _edge_call(
            gs, gd, e,
            w1[:ND], w1[ND:2 * ND], w1[2 * ND:2 * ND + ED], w1[2 * ND + ED:],
            b1.reshape(1, H), w2, b2.reshape(1, H), wx1, bx1.reshape(1, H),
            jnp.pad(wx2, ((0, 0), (0, 7))), jnp.pad(bx2, (0, 7)).reshape(1, 8),
            weu[:ED], weu[ED:], beu.reshape(1, ED))
        pm, pa = _seg_scatter_m(scat_m, scat_a, sidx, z_m, z_aux)
        tab = _node_call(tab, pm[0, :N], pm[1, :N], pa[0, :N], pa[1, :N],
                         wh1[:ND], wh1[ND:], bh1.reshape(1, H),
                         wh2, bh2.reshape(1, ND))

    o1, ob1 = params['out1']
    o2, ob2 = params['out2']
    gs, gd = _gather(tab, gsrc, gdst)
    pvec, ohrows, hidx = _final_call(
        gs, gd, e, src_s.reshape(E_PAD, 1), dst_g.reshape(E_PAD, 1),
        o1[:ND], o1[ND:2 * ND], o1[2 * ND:], ob1.reshape(1, H),
        jnp.pad(o2, ((0, 0), (0, 7))),
        jnp.pad(ob2, (0, 7)).reshape(1, 8))
    den, = _seg_scatter_p(pvec, sidx_h, z_den)
    hidx_r = jnp.stack([hidx[:, k].reshape(NS, NBH, BURST)
                        for k in range(NC * HPASSES)])
    hidx_r = hidx_r.reshape(NC, HPASSES, NS, NBH, BURST)
    hm = _heat(ohrows, hidx_r, z_heat)
    hm2d = hm[:, :HHALF, :].reshape(N, N)
    return _divide_call(hm2d, den[:N, 0:1])
